# Initial kernel scaffold; baseline (speedup 1.0000x reference)
import jax
import jax.numpy as jnp
from jax.experimental import pallas as pl


def kernel(node_features, step_indicators, edge_index, edge_to_nodes, params):
    raise NotImplementedError("write your pallas kernel here")


# speedup vs baseline: 1.0650x; 1.0650x over previous
import jax
import jax.numpy as jnp
from jax.experimental import pallas as pl
from jax.experimental.pallas import tpu as pltpu

HID = 256
HEADS = 8
HD = 32
LAYERS = 3


def _mm_kernel(x_ref, w_ref, o_ref):
    o_ref[...] = jnp.dot(x_ref[...], w_ref[...], preferred_element_type=jnp.float32)


def _mm(x, w, bm=2048):
    m, k = x.shape
    k2, n = w.shape
    if m <= bm:
        return pl.pallas_call(
            _mm_kernel,
            out_shape=jax.ShapeDtypeStruct((m, n), jnp.float32),
        )(x, w)
    grid = (pl.cdiv(m, bm),)
    return pl.pallas_call(
        _mm_kernel,
        grid=grid,
        in_specs=[
            pl.BlockSpec((bm, k), lambda i: (i, 0)),
            pl.BlockSpec((k, n), lambda i: (0, 0)),
        ],
        out_specs=pl.BlockSpec((bm, n), lambda i: (i, 0)),
        out_shape=jax.ShapeDtypeStruct((m, n), jnp.float32),
    )(x, w)


def _layernorm(x, g, b, eps=1e-5):
    mu = jnp.mean(x, axis=-1, keepdims=True)
    var = jnp.mean((x - mu) ** 2, axis=-1, keepdims=True)
    return (x - mu) / jnp.sqrt(var + eps) * g + b


def _gnn_layer(h, src, dst, p):
    Nn = h.shape[0]
    projected = _mm(h, p['W'])
    hh = projected.reshape(Nn, HEADS, HD)
    alpha_src_n = jnp.sum(hh * p['a_src'], axis=-1)
    alpha_dst_n = jnp.sum(hh * p['a_dst'], axis=-1)
    alpha = alpha_src_n[src] + alpha_dst_n[dst]
    alpha = jax.nn.leaky_relu(alpha, negative_slope=0.2)
    alpha = alpha - jnp.max(alpha, axis=0, keepdims=True)
    alpha = jnp.exp(alpha)
    alpha_sum = jnp.zeros((Nn, HEADS), jnp.float32).at[dst].add(alpha)
    msg = hh[src] * alpha[..., None]
    out = jnp.zeros((Nn, HEADS, HD), jnp.float32).at[dst].add(msg)
    out = out / (alpha_sum[..., None] + 1e-8)
    out = out.reshape(Nn, HID)
    out = _layernorm(out + projected, p['g'], p['b'])
    return jax.nn.relu(out)


def kernel(node_features, step_indicators, edge_index, edge_to_nodes, params):
    B, Nn, _ = node_features.shape
    step = jnp.broadcast_to(step_indicators[:, None, :], (B, Nn, step_indicators.shape[-1]))
    x = jnp.concatenate([node_features, step], axis=-1)[0]
    h = _mm(x, params['W_enc']) + params['b_enc']
    src = edge_index[0]
    dst = edge_index[1]
    for l in range(LAYERS):
        h = _gnn_layer(h, src, dst, params['layers'][l])
    s_hid = jax.nn.relu(_mm(h, params['W_s1']) + params['b_s1'])
    settlement_logits = (_mm(s_hid, params['W_s2']) + params['b_s2'])[None, :, 0]
    src_nodes = edge_to_nodes[:, 0]
    dst_nodes = edge_to_nodes[:, 1]
    Ps = _mm(h, params['W_r1'][:HID])
    Pd = _mm(h, params['W_r1'][HID:])
    r_hid = jax.nn.relu(Ps[src_nodes] + Pd[dst_nodes] + params['b_r1'])
    road_logits = (_mm(r_hid, params['W_r2']) + params['b_r2'])[None, :, 0]
    pooled = h.mean(axis=0, keepdims=True)
    v_hid = jax.nn.relu(_mm(pooled, params['W_v1']) + params['b_v1'])
    state_value = jnp.tanh(_mm(v_hid, params['W_v2']) + params['b_v2'])
    return (settlement_logits, road_logits, state_value)
